# baseline (speedup 1.0000x reference)
import functools

import jax
import jax.numpy as jnp
from jax import lax
from jax.experimental import pallas as pl
from jax.experimental.pallas import tpu as pltpu
from jax.experimental.pallas import tpu_sc as plsc

NPIX = 147456
NT = 16
CHUNK = NPIX // NT
NB16 = CHUNK // 16
NNZ = 368640
EPT = NNZ // NT
CM_G = 20
CM_E = EPT // CM_G
NLOC = 18432
LPT = NLOC // NT
LC = LPT // 128
NGH = LPT * 9
NIU = LPT * 6
STEPS = 30

_PAIRS = [(i, j) for i in range(9) for j in range(i + 1, 9)]


def _mesh():
    return plsc.VectorSubcoreMesh(
        core_axis_name="c", subcore_axis_name="s", num_cores=1, num_subcores=NT
    )


def _fill(ref, n16, value):
    v = jnp.full((16,), value, jnp.float32)

    def body(i, _):
        ref[pl.ds(i * 16, 16)] = v
        return 0

    lax.fori_loop(0, n16, body, 0, unroll=8)


def _cg_kernel(cmw_h, locw_h, iuw_h, diag_h, b_h,
               cmi_h, cmd_h,
               ngh_h, locin_h, ssym_h,
               iui_h, iuv_h, iuin_h,
               xout_h, vals_h,
               p_v, x_v, r_v, y_v, dcm_v, dd_v,
               ngh_v, wts_v, wiu_v, iuiv,
               ciA, ciB, cvA, cvB, pg_v,
               s45_v, pgm_v, om_v,
               red_v, pw_v,
               semA, semB,
               p_s, acc1_s, accw_s, red_s, red2_s):
    sid = lax.axis_index("s")
    base = sid * CHUNK
    sl = pl.ds(base, CHUNK)

    stg = pgm_v.at[pl.ds(0, CHUNK)]

    def stage_to_shared(src_h):
        pltpu.sync_copy(src_h.at[sl], stg)
        pltpu.sync_copy(stg, p_s.at[sl])

    def zero_shared(acc):
        pltpu.sync_copy(stg, acc.at[sl])

    def ew_loop(body):
        lax.fori_loop(0, NB16, body, 0, unroll=8)

    def finish_dot(acc16, red_sh):
        pw_v[...] = acc16
        pltpu.sync_copy(pw_v, red_sh.at[pl.ds(sid * 16, 16)])
        plsc.subcore_barrier()
        pltpu.sync_copy(red_sh, red_v)
        tot = red_v[pl.ds(0, 16)]
        for t in range(1, NT):
            tot = tot + red_v[pl.ds(t * 16, 16)]
        s = tot[0]
        for i in range(1, 16):
            s = s + tot[i]
        return s

    def cm_start(g, ci, cv, sem):
        pltpu.async_copy(cmi_h.at[sid, g], ci, sem)
        pltpu.async_copy(vals_h.at[sid, g], cv, sem)

    def cm_wait(g, ci, cv, sem):
        pltpu.make_async_copy(cmi_h.at[sid, g], ci, sem).wait()
        pltpu.make_async_copy(vals_h.at[sid, g], cv, sem).wait()

    def cm_work(ci, cv, go, so, src_s, acc_s):
        pltpu.sync_copy(src_s.at[ci.at[pl.ds(go, CM_E)]], pg_v)

        def mul(k, _):
            s16 = pl.ds(k * 16, 16)
            pg_v[s16] = cv[s16] * pg_v[s16]
            return 0

        lax.fori_loop(0, CM_E // 16, mul, 0, unroll=8)
        pltpu.sync_copy(pg_v, acc_s.at[ci.at[pl.ds(so, CM_E)]], add=True)

    def lcm_pass(go, so, src_s, acc_s):
        cm_start(0, ciA, cvA, semA)

        def pair(h, _):
            g = h * 2
            cm_wait(g, ciA, cvA, semA)
            cm_start(g + 1, ciB, cvB, semB)
            cm_work(ciA, cvA, go, so, src_s, acc_s)
            cm_wait(g + 1, ciB, cvB, semB)

            @pl.when(h < CM_G // 2 - 1)
            def _():
                cm_start(g + 2, ciA, cvA, semA)

            cm_work(ciB, cvB, go, so, src_s, acc_s)
            return 0

        lax.fori_loop(0, CM_G // 2, pair, 0)

    def lmat_pass(src_s, acc_s):
        def scbody(sc, _):
            pltpu.sync_copy(src_s.at[ngh_v.at[pl.ds(sc * 3456, 3456)]],
                            pgm_v.at[pl.ds(sc * 3456, 3456)])

            def cbody(c3, _):
                c = sc * 3 + c3
                pltpu.sync_copy(ssym_h.at[sid, c], s45_v)
                cb = c * 1152
                ob = c3 * 1152
                for k in range(8):
                    k16 = k * 16
                    wloc = wts_v[pl.ds(c * 128 + k16, 16)]
                    pgw = [pgm_v[pl.ds(cb + j * 128 + k16, 16)] * wloc
                           for j in range(9)]
                    o = [s45_v[pl.ds(i * 128 + k16, 16)] * pgw[i]
                         for i in range(9)]
                    for n, (i, j) in enumerate(_PAIRS):
                        t = s45_v[pl.ds((9 + n) * 128 + k16, 16)]
                        o[i] = o[i] + t * pgw[j]
                        o[j] = o[j] + t * pgw[i]
                    for i in range(9):
                        om_v[pl.ds(ob + i * 128 + k16, 16)] = o[i]
                return 0

            lax.fori_loop(0, 3, cbody, 0)
            pltpu.sync_copy(
                om_v, acc_s.at[ngh_v.at[pl.ds(sc * 3456, 3456)]], add=True)
            return 0

        lax.fori_loop(0, 3, scbody, 0)

    def iu_pass(src_s, acc_s):
        pltpu.sync_copy(iuv_h.at[sid], s45_v)

        def scbody(sc, _):
            pltpu.sync_copy(src_s.at[iuiv.at[pl.ds(sc * 2304, 2304)]],
                            pgm_v.at[pl.ds(sc * 2304, 2304)])

            def cbody(c3, _):
                c = sc * 3 + c3
                cb6 = c * 768
                ob6 = c3 * 768
                cb5 = c * 640
                for k in range(8):
                    k16 = k * 16
                    wloc = 0.5 * wiu_v[pl.ds(c * 128 + k16, 16)]
                    g0 = pgm_v[pl.ds(cb6 + k16, 16)]
                    ssum = (s45_v[pl.ds(cb5 + k16, 16)]
                            * pgm_v[pl.ds(cb6 + 128 + k16, 16)])
                    for j in range(1, 5):
                        ssum = ssum + (s45_v[pl.ds(cb5 + j * 128 + k16, 16)]
                                       * pgm_v[pl.ds(cb6 + (1 + j) * 128 + k16, 16)])
                    om_v[pl.ds(ob6 + k16, 16)] = wloc * ssum
                    wg0 = wloc * g0
                    for j in range(5):
                        om_v[pl.ds(ob6 + (1 + j) * 128 + k16, 16)] = (
                            s45_v[pl.ds(cb5 + j * 128 + k16, 16)] * wg0)
                return 0

            lax.fori_loop(0, 3, cbody, 0)
            pltpu.sync_copy(
                om_v.at[pl.ds(0, 2304)],
                acc_s.at[iuiv.at[pl.ds(sc * 2304, 2304)]], add=True)
            return 0

        lax.fori_loop(0, 3, scbody, 0)

    def ldpart(i, _):
        pltpu.sync_copy(ngh_h.at[sid, i], ngh_v.at[pl.ds(i * 3456, 3456)])
        pltpu.sync_copy(iui_h.at[sid, i], iuiv.at[pl.ds(i * 2304, 2304)])
        return 0

    lax.fori_loop(0, 3, ldpart, 0)
    _fill(pgm_v, NB16, 0.0)
    zero_shared(acc1_s)
    zero_shared(accw_s)
    stage_to_shared(cmw_h)
    plsc.subcore_barrier()

    def p1body(g, _):
        pltpu.sync_copy(cmi_h.at[sid, g], ciA)
        pltpu.sync_copy(cmd_h.at[sid, g], cvA)
        pltpu.sync_copy(p_s.at[ciA.at[pl.ds(0, CM_E)]], pg_v)

        def mul(k, _):
            s16 = pl.ds(k * 16, 16)
            pg_v[s16] = cvA[s16] * pg_v[s16]
            return 0

        lax.fori_loop(0, CM_E // 16, mul, 0, unroll=8)
        pltpu.sync_copy(pg_v, acc1_s.at[ciA.at[pl.ds(0, CM_E)]], add=True)
        pltpu.sync_copy(pg_v, vals_h.at[sid, g])
        return 0

    lax.fori_loop(0, CM_G, p1body, 0)
    plsc.subcore_barrier()
    pltpu.sync_copy(acc1_s.at[sl], dcm_v)

    stage_to_shared(locw_h)
    plsc.subcore_barrier()
    pltpu.sync_copy(locin_h.at[sid], ciA.at[pl.ds(0, LPT)])
    pltpu.sync_copy(p_s.at[ciA.at[pl.ds(0, LPT)]], wts_v)
    plsc.subcore_barrier()
    stage_to_shared(iuw_h)
    plsc.subcore_barrier()
    pltpu.sync_copy(iuin_h.at[sid], ciA.at[pl.ds(0, LPT)])
    pltpu.sync_copy(p_s.at[ciA.at[pl.ds(0, LPT)]], wiu_v)
    plsc.subcore_barrier()

    _fill(pgm_v, NB16, 1.0)
    pltpu.sync_copy(stg, p_s.at[sl])
    plsc.subcore_barrier()
    lmat_pass(p_s, accw_s)
    iu_pass(p_s, accw_s)
    plsc.subcore_barrier()
    pltpu.sync_copy(accw_s.at[sl], stg)
    pltpu.sync_copy(diag_h.at[sl], dd_v)

    def ddbody(i, _):
        s16 = pl.ds(i * 16, 16)
        dd_v[s16] = dd_v[s16] + pgm_v[s16]
        return 0

    ew_loop(ddbody)

    pltpu.sync_copy(b_h.at[sl], r_v)
    zero16 = jnp.zeros((16,), jnp.float32)

    def initbody(i, acc):
        s16 = pl.ds(i * 16, 16)
        rv = r_v[s16]
        p_v[s16] = rv
        x_v[s16] = zero16
        return acc + rv * rv

    rs0v = lax.fori_loop(0, NB16, initbody, zero16, unroll=8)
    rs0 = finish_dot(rs0v, red2_s)

    def step(_, rs):
        _fill(pgm_v, NB16, 0.0)
        pltpu.sync_copy(p_v, p_s.at[sl])
        zero_shared(acc1_s)
        zero_shared(accw_s)
        plsc.subcore_barrier()
        lcm_pass(CM_E, 0, p_s, acc1_s)
        lmat_pass(p_s, accw_s)
        iu_pass(p_s, accw_s)
        plsc.subcore_barrier()
        pltpu.sync_copy(acc1_s.at[sl], stg)

        def ybody(i, _):
            s16 = pl.ds(i * 16, 16)
            y_v[s16] = dcm_v[s16] * p_v[s16] - pgm_v[s16]
            return 0

        ew_loop(ybody)
        pltpu.sync_copy(y_v, p_s.at[sl])
        _fill(pgm_v, NB16, 0.0)
        zero_shared(acc1_s)
        plsc.subcore_barrier()
        lcm_pass(0, CM_E, p_s, acc1_s)
        plsc.subcore_barrier()
        pltpu.sync_copy(acc1_s.at[sl], stg)

        def ap1(i, _):
            s16 = pl.ds(i * 16, 16)
            y_v[s16] = dcm_v[s16] * y_v[s16] - pgm_v[s16]
            return 0

        ew_loop(ap1)
        pltpu.sync_copy(accw_s.at[sl], stg)

        def ap2(i, acc):
            s16 = pl.ds(i * 16, 16)
            ap = y_v[s16] + dd_v[s16] * p_v[s16] - pgm_v[s16]
            y_v[s16] = ap
            return acc + p_v[s16] * ap

        papv = lax.fori_loop(0, NB16, ap2, zero16, unroll=8)
        pap = finish_dot(papv, red_s)
        alpha = jnp.broadcast_to(rs, (16,)) / (
            jnp.broadcast_to(pap, (16,)) + 1e-12)

        def upd(i, acc):
            s16 = pl.ds(i * 16, 16)
            x_v[s16] = x_v[s16] + alpha * p_v[s16]
            rv = r_v[s16] - alpha * y_v[s16]
            r_v[s16] = rv
            return acc + rv * rv

        rrv = lax.fori_loop(0, NB16, upd, zero16, unroll=8)
        rr = finish_dot(rrv, red2_s)
        beta = jnp.broadcast_to(rr, (16,)) / (
            jnp.broadcast_to(rs, (16,)) + 1e-12)

        def pupd(i, _):
            s16 = pl.ds(i * 16, 16)
            p_v[s16] = r_v[s16] + beta * p_v[s16]
            return 0

        ew_loop(pupd)
        return rr

    lax.fori_loop(0, STEPS, step, rs0)
    pltpu.sync_copy(x_v, xout_h.at[sl])


@jax.jit
def _run(cmw, locw, iuw, diag, b, cmi, cmd, ngh, locin, ssym, iui, iuv,
         iuin):
    f = pl.kernel(
        _cg_kernel,
        out_type=[
            jax.ShapeDtypeStruct((NPIX,), jnp.float32),
            jax.ShapeDtypeStruct((NT, CM_G, CM_E), jnp.float32),
        ],
        mesh=_mesh(),
        scratch_types=[
            pltpu.VMEM((CHUNK,), jnp.float32),
            pltpu.VMEM((CHUNK,), jnp.float32),
            pltpu.VMEM((CHUNK,), jnp.float32),
            pltpu.VMEM((CHUNK,), jnp.float32),
            pltpu.VMEM((CHUNK,), jnp.float32),
            pltpu.VMEM((CHUNK,), jnp.float32),
            pltpu.VMEM((NGH,), jnp.int32),
            pltpu.VMEM((LPT,), jnp.float32),
            pltpu.VMEM((LPT,), jnp.float32),
            pltpu.VMEM((NIU,), jnp.int32),
            pltpu.VMEM((2 * CM_E,), jnp.int32),
            pltpu.VMEM((2 * CM_E,), jnp.int32),
            pltpu.VMEM((CM_E,), jnp.float32),
            pltpu.VMEM((CM_E,), jnp.float32),
            pltpu.VMEM((CM_E,), jnp.float32),
            pltpu.VMEM((45 * 128,), jnp.float32),
            pltpu.VMEM((NGH,), jnp.float32),
            pltpu.VMEM((3456,), jnp.float32),
            pltpu.VMEM((NT * 16,), jnp.float32),
            pltpu.VMEM((16,), jnp.float32),
            pltpu.SemaphoreType.DMA,
            pltpu.SemaphoreType.DMA,
            pltpu.VMEM_SHARED((NPIX,), jnp.float32),
            pltpu.VMEM_SHARED((NPIX,), jnp.float32),
            pltpu.VMEM_SHARED((NPIX,), jnp.float32),
            pltpu.VMEM_SHARED((NT * 16,), jnp.float32),
            pltpu.VMEM_SHARED((NT * 16,), jnp.float32),
        ],
    )
    x, _ = f(cmw, locw, iuw, diag, b, cmi, cmd, ngh, locin, ssym, iui,
             iuv, iuin)
    return x


def kernel(CM_weights, LOC_weights, IU_weights, KU_weights, lmbda, kToUconf,
           known, kToU, Wcm_row, Wcm_col, Wcm_data, LOC_inInd, LOC_flows,
           IU_inInd, IU_neighInd, IU_flows, N, w):
    Nn = CM_weights.shape[0]
    cmr = Wcm_row.reshape(NT, CM_G, CM_E)
    cmc = Wcm_col.reshape(NT, CM_G, CM_E)
    cmi = jnp.concatenate([cmr, cmc], axis=-1)
    cmd = Wcm_data.reshape(NT, CM_G, CM_E)

    inInd = LOC_inInd.reshape(-1)
    wi = jnp.asarray(w, jnp.int32)
    offs = jnp.stack([-1 - wi, -jnp.ones((), jnp.int32), -1 + wi, -wi,
                      jnp.zeros((), jnp.int32), wi, 1 - wi,
                      jnp.ones((), jnp.int32), 1 + wi])
    neigh = jnp.clip(inInd[:, None] + offs[None, :], 0, Nn - 1)
    ngh = neigh.reshape(NT, LC, 128, 9).transpose(0, 1, 3, 2).reshape(
        NT, 3, NGH // 3)
    locin = inInd.reshape(NT, LPT)
    ssym = 0.5 * (LOC_flows + LOC_flows.transpose(1, 0, 2))
    rows = [ssym[i, i] for i in range(9)] + [ssym[i, j] for i, j in _PAIRS]
    s45 = jnp.stack(rows)
    s45 = s45.reshape(45, NT, LC, 128).transpose(1, 2, 0, 3).reshape(
        NT, LC, 45 * 128)

    iu = IU_inInd.reshape(-1)
    iui = jnp.concatenate([iu[:, None], IU_neighInd], axis=1)
    iui = iui.reshape(NT, LC, 128, 6).transpose(0, 1, 3, 2).reshape(
        NT, 3, NIU // 3)
    iuv = IU_flows.reshape(NT, LC, 128, 5).transpose(0, 1, 3, 2).reshape(
        NT, LC * 640)
    iuin = iu.reshape(NT, LPT)

    diag = KU_weights * kToUconf + lmbda[0] * known
    b = diag * kToU

    return _run(CM_weights, LOC_weights, IU_weights, diag, b,
                cmi, cmd, ngh, locin, s45, iui, iuv, iuin)


# speedup vs baseline: 1.0159x; 1.0159x over previous
import functools

import jax
import jax.numpy as jnp
from jax import lax
from jax.experimental import pallas as pl
from jax.experimental.pallas import tpu as pltpu
from jax.experimental.pallas import tpu_sc as plsc

NPIX = 147456
NT = 16
CHUNK = NPIX // NT
NB16 = CHUNK // 16
NNZ = 368640
EPT = NNZ // NT
CM_G = 20
CM_E = EPT // CM_G
NLOC = 18432
LPT = NLOC // NT
LC = LPT // 128
NGH = LPT * 9
NIU = LPT * 6
STEPS = 30

_PAIRS = [(i, j) for i in range(9) for j in range(i + 1, 9)]


def _mesh():
    return plsc.VectorSubcoreMesh(
        core_axis_name="c", subcore_axis_name="s", num_cores=1, num_subcores=NT
    )


def _fill(ref, n16, value):
    v = jnp.full((16,), value, jnp.float32)

    def body(i, _):
        ref[pl.ds(i * 16, 16)] = v
        return 0

    lax.fori_loop(0, n16, body, 0, unroll=8)


def _cg_kernel(cmw_h, locw_h, iuw_h, diag_h, b_h,
               cmi_h, cmd_h,
               ngh_h, locin_h, ssym_h,
               iui_h, iuv_h, iuin_h,
               xout_h, vals_h,
               p_v, x_v, r_v, y_v, dcm_v, dd_v,
               ngh_v, wts_v, wiu_v, iuiv,
               ciA, ciB, cvA, cvB, pg_v,
               s45_v, pgm_v, om_v,
               red_v, pw_v,
               semA, semB,
               p_s, acc1_s, accw_s, red_s, red2_s):
    sid = lax.axis_index("s")
    base = sid * CHUNK
    sl = pl.ds(base, CHUNK)

    stg = pgm_v.at[pl.ds(0, CHUNK)]

    def stage_to_shared(src_h):
        pltpu.sync_copy(src_h.at[sl], stg)
        pltpu.sync_copy(stg, p_s.at[sl])

    def zero_shared(acc):
        pltpu.sync_copy(stg, acc.at[sl])

    def ew_loop(body):
        lax.fori_loop(0, NB16, body, 0, unroll=8)

    def finish_dot(acc16, red_sh):
        pw_v[...] = acc16
        pltpu.sync_copy(pw_v, red_sh.at[pl.ds(sid * 16, 16)])
        plsc.subcore_barrier()
        pltpu.sync_copy(red_sh, red_v)
        tot = red_v[pl.ds(0, 16)]
        for t in range(1, NT):
            tot = tot + red_v[pl.ds(t * 16, 16)]
        s = tot[0]
        for i in range(1, 16):
            s = s + tot[i]
        return s

    def cm_start(g, ci, cv, sem):
        pltpu.async_copy(cmi_h.at[sid, g], ci, sem)
        pltpu.async_copy(vals_h.at[sid, g], cv, sem)

    def cm_wait(g, ci, cv, sem):
        pltpu.make_async_copy(cmi_h.at[sid, g], ci, sem).wait()
        pltpu.make_async_copy(vals_h.at[sid, g], cv, sem).wait()

    def cm_work(ci, cv, go, so, src_s, acc_s):
        pltpu.sync_copy(src_s.at[ci.at[pl.ds(go, CM_E)]], pg_v)

        def mul(k, _):
            s16 = pl.ds(k * 16, 16)
            pg_v[s16] = cv[s16] * pg_v[s16]
            return 0

        lax.fori_loop(0, CM_E // 16, mul, 0, unroll=8)
        pltpu.sync_copy(pg_v, acc_s.at[ci.at[pl.ds(so, CM_E)]], add=True)

    def lcm_pass(go, so, src_s, acc_s):
        cm_start(0, ciA, cvA, semA)

        def pair(h, _):
            g = h * 2
            cm_wait(g, ciA, cvA, semA)
            cm_start(g + 1, ciB, cvB, semB)
            cm_work(ciA, cvA, go, so, src_s, acc_s)
            cm_wait(g + 1, ciB, cvB, semB)

            @pl.when(h < CM_G // 2 - 1)
            def _():
                cm_start(g + 2, ciA, cvA, semA)

            cm_work(ciB, cvB, go, so, src_s, acc_s)
            return 0

        lax.fori_loop(0, CM_G // 2, pair, 0)

    def lmat_pass(src_s, acc_s):
        def cbody(c, _):
                pltpu.sync_copy(src_s.at[ngh_v.at[pl.ds(c * 1152, 1152)]],
                                pgm_v.at[pl.ds(c * 1152, 1152)])
                pltpu.sync_copy(ssym_h.at[sid, c], s45_v)
                cb = c * 1152
                ob = 0
                for k in range(8):
                    k16 = k * 16
                    wloc = wts_v[pl.ds(c * 128 + k16, 16)]
                    pgw = [pgm_v[pl.ds(cb + j * 128 + k16, 16)] * wloc
                           for j in range(9)]
                    o = [s45_v[pl.ds(i * 128 + k16, 16)] * pgw[i]
                         for i in range(9)]
                    for n, (i, j) in enumerate(_PAIRS):
                        t = s45_v[pl.ds((9 + n) * 128 + k16, 16)]
                        o[i] = o[i] + t * pgw[j]
                        o[j] = o[j] + t * pgw[i]
                    for i in range(9):
                        om_v[pl.ds(ob + i * 128 + k16, 16)] = o[i]
                pltpu.sync_copy(
                    om_v.at[pl.ds(0, 1152)],
                    acc_s.at[ngh_v.at[pl.ds(c * 1152, 1152)]], add=True)
                return 0

        lax.fori_loop(0, LC, cbody, 0)

    def iu_pass(src_s, acc_s):
        pltpu.sync_copy(iuv_h.at[sid], s45_v)

        def cbody(c, _):
                pltpu.sync_copy(src_s.at[iuiv.at[pl.ds(c * 768, 768)]],
                                pgm_v.at[pl.ds(c * 768, 768)])
                cb6 = c * 768
                ob6 = 0
                cb5 = c * 640
                for k in range(8):
                    k16 = k * 16
                    wloc = 0.5 * wiu_v[pl.ds(c * 128 + k16, 16)]
                    g0 = pgm_v[pl.ds(cb6 + k16, 16)]
                    ssum = (s45_v[pl.ds(cb5 + k16, 16)]
                            * pgm_v[pl.ds(cb6 + 128 + k16, 16)])
                    for j in range(1, 5):
                        ssum = ssum + (s45_v[pl.ds(cb5 + j * 128 + k16, 16)]
                                       * pgm_v[pl.ds(cb6 + (1 + j) * 128 + k16, 16)])
                    om_v[pl.ds(ob6 + k16, 16)] = wloc * ssum
                    wg0 = wloc * g0
                    for j in range(5):
                        om_v[pl.ds(ob6 + (1 + j) * 128 + k16, 16)] = (
                            s45_v[pl.ds(cb5 + j * 128 + k16, 16)] * wg0)
                pltpu.sync_copy(
                    om_v.at[pl.ds(0, 768)],
                    acc_s.at[iuiv.at[pl.ds(c * 768, 768)]], add=True)
                return 0

        lax.fori_loop(0, LC, cbody, 0)

    def ldpart(i, _):
        pltpu.sync_copy(ngh_h.at[sid, i], ngh_v.at[pl.ds(i * 3456, 3456)])
        pltpu.sync_copy(iui_h.at[sid, i], iuiv.at[pl.ds(i * 2304, 2304)])
        return 0

    lax.fori_loop(0, 3, ldpart, 0)
    _fill(pgm_v, NB16, 0.0)
    zero_shared(acc1_s)
    zero_shared(accw_s)
    stage_to_shared(cmw_h)
    plsc.subcore_barrier()

    def p1body(g, _):
        pltpu.sync_copy(cmi_h.at[sid, g], ciA)
        pltpu.sync_copy(cmd_h.at[sid, g], cvA)
        pltpu.sync_copy(p_s.at[ciA.at[pl.ds(0, CM_E)]], pg_v)

        def mul(k, _):
            s16 = pl.ds(k * 16, 16)
            pg_v[s16] = cvA[s16] * pg_v[s16]
            return 0

        lax.fori_loop(0, CM_E // 16, mul, 0, unroll=8)
        pltpu.sync_copy(pg_v, acc1_s.at[ciA.at[pl.ds(0, CM_E)]], add=True)
        pltpu.sync_copy(pg_v, vals_h.at[sid, g])
        return 0

    lax.fori_loop(0, CM_G, p1body, 0)
    plsc.subcore_barrier()
    pltpu.sync_copy(acc1_s.at[sl], dcm_v)

    stage_to_shared(locw_h)
    plsc.subcore_barrier()
    pltpu.sync_copy(locin_h.at[sid], ciA.at[pl.ds(0, LPT)])
    pltpu.sync_copy(p_s.at[ciA.at[pl.ds(0, LPT)]], wts_v)
    plsc.subcore_barrier()
    stage_to_shared(iuw_h)
    plsc.subcore_barrier()
    pltpu.sync_copy(iuin_h.at[sid], ciA.at[pl.ds(0, LPT)])
    pltpu.sync_copy(p_s.at[ciA.at[pl.ds(0, LPT)]], wiu_v)
    plsc.subcore_barrier()

    _fill(pgm_v, NB16, 1.0)
    pltpu.sync_copy(stg, p_s.at[sl])
    plsc.subcore_barrier()
    lmat_pass(p_s, accw_s)
    iu_pass(p_s, accw_s)
    plsc.subcore_barrier()
    pltpu.sync_copy(accw_s.at[sl], stg)
    pltpu.sync_copy(diag_h.at[sl], dd_v)

    def ddbody(i, _):
        s16 = pl.ds(i * 16, 16)
        dd_v[s16] = dd_v[s16] + pgm_v[s16]
        return 0

    ew_loop(ddbody)

    pltpu.sync_copy(b_h.at[sl], r_v)
    zero16 = jnp.zeros((16,), jnp.float32)

    def initbody(i, acc):
        s16 = pl.ds(i * 16, 16)
        rv = r_v[s16]
        p_v[s16] = rv
        x_v[s16] = zero16
        return acc + rv * rv

    rs0v = lax.fori_loop(0, NB16, initbody, zero16, unroll=8)
    rs0 = finish_dot(rs0v, red2_s)

    def step(_, rs):
        _fill(pgm_v, NB16, 0.0)
        pltpu.sync_copy(p_v, p_s.at[sl])
        zero_shared(acc1_s)
        zero_shared(accw_s)
        plsc.subcore_barrier()
        lcm_pass(CM_E, 0, p_s, acc1_s)
        lmat_pass(p_s, accw_s)
        iu_pass(p_s, accw_s)
        plsc.subcore_barrier()
        pltpu.sync_copy(acc1_s.at[sl], stg)

        def ybody(i, _):
            s16 = pl.ds(i * 16, 16)
            y_v[s16] = dcm_v[s16] * p_v[s16] - pgm_v[s16]
            return 0

        ew_loop(ybody)
        pltpu.sync_copy(y_v, p_s.at[sl])
        _fill(pgm_v, NB16, 0.0)
        zero_shared(acc1_s)
        plsc.subcore_barrier()
        lcm_pass(0, CM_E, p_s, acc1_s)
        plsc.subcore_barrier()
        pltpu.sync_copy(acc1_s.at[sl], stg)

        def ap1(i, _):
            s16 = pl.ds(i * 16, 16)
            y_v[s16] = dcm_v[s16] * y_v[s16] - pgm_v[s16]
            return 0

        ew_loop(ap1)
        pltpu.sync_copy(accw_s.at[sl], stg)

        def ap2(i, acc):
            s16 = pl.ds(i * 16, 16)
            ap = y_v[s16] + dd_v[s16] * p_v[s16] - pgm_v[s16]
            y_v[s16] = ap
            return acc + p_v[s16] * ap

        papv = lax.fori_loop(0, NB16, ap2, zero16, unroll=8)
        pap = finish_dot(papv, red_s)
        alpha = jnp.broadcast_to(rs, (16,)) / (
            jnp.broadcast_to(pap, (16,)) + 1e-12)

        def upd(i, acc):
            s16 = pl.ds(i * 16, 16)
            x_v[s16] = x_v[s16] + alpha * p_v[s16]
            rv = r_v[s16] - alpha * y_v[s16]
            r_v[s16] = rv
            return acc + rv * rv

        rrv = lax.fori_loop(0, NB16, upd, zero16, unroll=8)
        rr = finish_dot(rrv, red2_s)
        beta = jnp.broadcast_to(rr, (16,)) / (
            jnp.broadcast_to(rs, (16,)) + 1e-12)

        def pupd(i, _):
            s16 = pl.ds(i * 16, 16)
            p_v[s16] = r_v[s16] + beta * p_v[s16]
            return 0

        ew_loop(pupd)
        return rr

    lax.fori_loop(0, STEPS, step, rs0)
    pltpu.sync_copy(x_v, xout_h.at[sl])


@jax.jit
def _run(cmw, locw, iuw, diag, b, cmi, cmd, ngh, locin, ssym, iui, iuv,
         iuin):
    f = pl.kernel(
        _cg_kernel,
        out_type=[
            jax.ShapeDtypeStruct((NPIX,), jnp.float32),
            jax.ShapeDtypeStruct((NT, CM_G, CM_E), jnp.float32),
        ],
        mesh=_mesh(),
        scratch_types=[
            pltpu.VMEM((CHUNK,), jnp.float32),
            pltpu.VMEM((CHUNK,), jnp.float32),
            pltpu.VMEM((CHUNK,), jnp.float32),
            pltpu.VMEM((CHUNK,), jnp.float32),
            pltpu.VMEM((CHUNK,), jnp.float32),
            pltpu.VMEM((CHUNK,), jnp.float32),
            pltpu.VMEM((NGH,), jnp.int32),
            pltpu.VMEM((LPT,), jnp.float32),
            pltpu.VMEM((LPT,), jnp.float32),
            pltpu.VMEM((NIU,), jnp.int32),
            pltpu.VMEM((2 * CM_E,), jnp.int32),
            pltpu.VMEM((2 * CM_E,), jnp.int32),
            pltpu.VMEM((CM_E,), jnp.float32),
            pltpu.VMEM((CM_E,), jnp.float32),
            pltpu.VMEM((CM_E,), jnp.float32),
            pltpu.VMEM((45 * 128,), jnp.float32),
            pltpu.VMEM((NGH,), jnp.float32),
            pltpu.VMEM((1152,), jnp.float32),
            pltpu.VMEM((NT * 16,), jnp.float32),
            pltpu.VMEM((16,), jnp.float32),
            pltpu.SemaphoreType.DMA,
            pltpu.SemaphoreType.DMA,
            pltpu.VMEM_SHARED((NPIX,), jnp.float32),
            pltpu.VMEM_SHARED((NPIX,), jnp.float32),
            pltpu.VMEM_SHARED((NPIX,), jnp.float32),
            pltpu.VMEM_SHARED((NT * 16,), jnp.float32),
            pltpu.VMEM_SHARED((NT * 16,), jnp.float32),
        ],
    )
    x, _ = f(cmw, locw, iuw, diag, b, cmi, cmd, ngh, locin, ssym, iui,
             iuv, iuin)
    return x


def kernel(CM_weights, LOC_weights, IU_weights, KU_weights, lmbda, kToUconf,
           known, kToU, Wcm_row, Wcm_col, Wcm_data, LOC_inInd, LOC_flows,
           IU_inInd, IU_neighInd, IU_flows, N, w):
    Nn = CM_weights.shape[0]
    cmr = Wcm_row.reshape(NT, CM_G, CM_E)
    cmc = Wcm_col.reshape(NT, CM_G, CM_E)
    cmi = jnp.concatenate([cmr, cmc], axis=-1)
    cmd = Wcm_data.reshape(NT, CM_G, CM_E)

    inInd = LOC_inInd.reshape(-1)
    wi = jnp.asarray(w, jnp.int32)
    offs = jnp.stack([-1 - wi, -jnp.ones((), jnp.int32), -1 + wi, -wi,
                      jnp.zeros((), jnp.int32), wi, 1 - wi,
                      jnp.ones((), jnp.int32), 1 + wi])
    neigh = jnp.clip(inInd[:, None] + offs[None, :], 0, Nn - 1)
    ngh = neigh.reshape(NT, LC, 128, 9).transpose(0, 1, 3, 2).reshape(
        NT, 3, NGH // 3)
    locin = inInd.reshape(NT, LPT)
    ssym = 0.5 * (LOC_flows + LOC_flows.transpose(1, 0, 2))
    rows = [ssym[i, i] for i in range(9)] + [ssym[i, j] for i, j in _PAIRS]
    s45 = jnp.stack(rows)
    s45 = s45.reshape(45, NT, LC, 128).transpose(1, 2, 0, 3).reshape(
        NT, LC, 45 * 128)

    iu = IU_inInd.reshape(-1)
    iui = jnp.concatenate([iu[:, None], IU_neighInd], axis=1)
    iui = iui.reshape(NT, LC, 128, 6).transpose(0, 1, 3, 2).reshape(
        NT, 3, NIU // 3)
    iuv = IU_flows.reshape(NT, LC, 128, 5).transpose(0, 1, 3, 2).reshape(
        NT, LC * 640)
    iuin = iu.reshape(NT, LPT)

    diag = KU_weights * kToUconf + lmbda[0] * known
    b = diag * kToU

    return _run(CM_weights, LOC_weights, IU_weights, diag, b,
                cmi, cmd, ngh, locin, s45, iui, iuv, iuin)
